# Initial kernel scaffold; baseline (speedup 1.0000x reference)
import jax
import jax.numpy as jnp
from jax.experimental import pallas as pl


def kernel(features, preference, mlp_W, mlp_b, conv1_W, lin1_W, lin1_b, g1_W, g1_b, conv2_W, lin2_W, lin2_b, g2_W, g2_b, id_embedding, edge_index):
    raise NotImplementedError("write your pallas kernel here")


# speedup vs baseline: 5.4375x; 5.4375x over previous
import functools

import jax
import jax.numpy as jnp
from jax import lax
from jax.experimental import pallas as pl
from jax.experimental.pallas import tpu as pltpu
from jax.experimental.pallas import tpu_sc as plsc

N_USERS = 25000
N_ITEMS = 25000
N_NODES = N_USERS + N_ITEMS
N_EDGES = 800000
D_FEAT = 128
D = 64
Q = 16

NSUB = 16
E_PER_SUB = N_EDGES // NSUB
CHUNK = 1000
R_BIG = 3128
R_LAST = N_NODES - (NSUB - 1) * R_BIG

BR = 1000


def _lrelu(x):
    return jnp.where(x > 0, x, 0.01 * x)


def _quarter_specs():
    return [pl.BlockSpec((BR, Q), lambda i: (i, 0)) for _ in range(4)]


def _quarter_shapes():
    return [jax.ShapeDtypeStruct((N_NODES, Q), jnp.float32) for _ in range(4)]


def _k1_body(f_ref, w_ref, b_ref, o_ref):
    o_ref[...] = jnp.dot(f_ref[...], w_ref[...],
                         preferred_element_type=jnp.float32) + b_ref[...]


def _mm_bias(features, mlp_W, mlp_b):
    return pl.pallas_call(
        _k1_body,
        grid=(N_ITEMS // BR,),
        in_specs=[
            pl.BlockSpec((BR, D_FEAT), lambda i: (i, 0)),
            pl.BlockSpec((D_FEAT, D), lambda i: (0, 0)),
            pl.BlockSpec((1, D), lambda i: (0, 0)),
        ],
        out_specs=pl.BlockSpec((BR, D), lambda i: (i, 0)),
        out_shape=jax.ShapeDtypeStruct((N_ITEMS, D), jnp.float32),
    )(features, mlp_W, mlp_b)


def _k2_body(x_ref, id_ref, cw_ref, lw_ref, lb_ref,
             q0_ref, q1_ref, q2_ref, q3_ref, xh_ref):
    x = x_ref[...]
    n = jnp.sqrt(jnp.sum(x * x, axis=1, keepdims=True))
    xn = x / jnp.maximum(n, 1e-12)
    xw = jnp.dot(xn, cw_ref[...], preferred_element_type=jnp.float32)
    q0_ref[...] = xw[:, 0 * Q:1 * Q]
    q1_ref[...] = xw[:, 1 * Q:2 * Q]
    q2_ref[...] = xw[:, 2 * Q:3 * Q]
    q3_ref[...] = xw[:, 3 * Q:4 * Q]
    xh_ref[...] = _lrelu(jnp.dot(xn, lw_ref[...],
                                 preferred_element_type=jnp.float32)
                         + lb_ref[...]) + id_ref[...]


def _layer_pre(x0, conv1_W, lin1_W, lin1_b, id_embedding):
    return pl.pallas_call(
        _k2_body,
        grid=(N_NODES // BR,),
        in_specs=[
            pl.BlockSpec((BR, D), lambda i: (i, 0)),
            pl.BlockSpec((BR, D), lambda i: (i, 0)),
            pl.BlockSpec((D, D), lambda i: (0, 0)),
            pl.BlockSpec((D, D), lambda i: (0, 0)),
            pl.BlockSpec((1, D), lambda i: (0, 0)),
        ],
        out_specs=_quarter_specs() + [pl.BlockSpec((BR, D), lambda i: (i, 0))],
        out_shape=_quarter_shapes() + [
            jax.ShapeDtypeStruct((N_NODES, D), jnp.float32)],
    )(x0, id_embedding, conv1_W, lin1_W, lin1_b)


def _k3_body(s0_ref, s1_ref, s2_ref, s3_ref, xh1_ref, id_ref, gw_ref, gb_ref,
             cw_ref, lw_ref, lb_ref, q0_ref, q1_ref, q2_ref, q3_ref, xh2_ref):
    h = _lrelu(jnp.concatenate(
        [s0_ref[...], s1_ref[...], s2_ref[...], s3_ref[...]], axis=1))
    x1 = _lrelu(jnp.dot(h, gw_ref[...], preferred_element_type=jnp.float32)
                + gb_ref[...] + xh1_ref[...])
    xw = jnp.dot(x1, cw_ref[...], preferred_element_type=jnp.float32)
    q0_ref[...] = xw[:, 0 * Q:1 * Q]
    q1_ref[...] = xw[:, 1 * Q:2 * Q]
    q2_ref[...] = xw[:, 2 * Q:3 * Q]
    q3_ref[...] = xw[:, 3 * Q:4 * Q]
    xh2_ref[...] = _lrelu(jnp.dot(x1, lw_ref[...],
                                  preferred_element_type=jnp.float32)
                          + lb_ref[...]) + id_ref[...]


def _layer_mid(segs, xhat1, id_embedding, g1_W, g1_b, conv2_W,
               lin2_W, lin2_b):
    return pl.pallas_call(
        _k3_body,
        grid=(N_NODES // BR,),
        in_specs=_quarter_specs() + [
            pl.BlockSpec((BR, D), lambda i: (i, 0)),
            pl.BlockSpec((BR, D), lambda i: (i, 0)),
            pl.BlockSpec((D, D), lambda i: (0, 0)),
            pl.BlockSpec((1, D), lambda i: (0, 0)),
            pl.BlockSpec((D, D), lambda i: (0, 0)),
            pl.BlockSpec((D, D), lambda i: (0, 0)),
            pl.BlockSpec((1, D), lambda i: (0, 0)),
        ],
        out_specs=_quarter_specs() + [pl.BlockSpec((BR, D), lambda i: (i, 0))],
        out_shape=_quarter_shapes() + [
            jax.ShapeDtypeStruct((N_NODES, D), jnp.float32)],
    )(*segs, xhat1, id_embedding, g1_W, g1_b, conv2_W, lin2_W, lin2_b)


def _k4_body(s0_ref, s1_ref, s2_ref, s3_ref, xh2_ref, gw_ref, gb_ref, o_ref):
    h = _lrelu(jnp.concatenate(
        [s0_ref[...], s1_ref[...], s2_ref[...], s3_ref[...]], axis=1))
    o_ref[...] = _lrelu(jnp.dot(h, gw_ref[...],
                                preferred_element_type=jnp.float32)
                        + gb_ref[...] + xh2_ref[...])


def _layer_post(segs, xhat2, g2_W, g2_b):
    return pl.pallas_call(
        _k4_body,
        grid=(N_NODES // BR,),
        in_specs=_quarter_specs() + [
            pl.BlockSpec((BR, D), lambda i: (i, 0)),
            pl.BlockSpec((D, D), lambda i: (0, 0)),
            pl.BlockSpec((1, D), lambda i: (0, 0)),
        ],
        out_specs=pl.BlockSpec((BR, D), lambda i: (i, 0)),
        out_shape=jax.ShapeDtypeStruct((N_NODES, D), jnp.float32),
    )(*segs, xhat2, g2_W, g2_b)


def _segsum(xq, src, dst, zrows):
    mesh = plsc.VectorSubcoreMesh(core_axis_name="c", subcore_axis_name="s")

    @functools.partial(
        pl.kernel,
        out_type=tuple(jax.ShapeDtypeStruct((N_NODES, Q), jnp.float32)
                       for _ in range(4)),
        mesh=mesh,
        scratch_types=[
            pltpu.VMEM((CHUNK,), jnp.int32),
            pltpu.VMEM((CHUNK,), jnp.int32),
            pltpu.VMEM((CHUNK, Q), jnp.float32),
            pltpu.VMEM_SHARED((N_NODES, Q), jnp.float32),
            pltpu.SemaphoreType.DMA,
        ],
        compiler_params=pltpu.CompilerParams(use_tc_tiling_on_sc=False),
    )
    def seg_kernel(x0_hbm, x1_hbm, x2_hbm, x3_hbm, src_hbm, dst_hbm, z_hbm,
                   o0_hbm, o1_hbm, o2_hbm, o3_hbm,
                   sidx_v, didx_v, rows_v, accum_sh, sem):
        c = lax.axis_index("c")
        s = lax.axis_index("s")
        r0 = pl.multiple_of(s * R_BIG, 8)

        def zero_slice():
            @pl.when(s < NSUB - 1)
            def _():
                pltpu.sync_copy(z_hbm, accum_sh.at[pl.ds(r0, R_BIG)])

            @pl.when(s == NSUB - 1)
            def _():
                pltpu.sync_copy(z_hbm.at[pl.ds(0, R_LAST)],
                                accum_sh.at[pl.ds(r0, R_LAST)])

        def run_edges(table_hbm):
            base_e = pl.multiple_of(s * E_PER_SUB, 8)

            @pl.loop(0, E_PER_SUB, step=CHUNK)
            def _(k):
                e0 = pl.multiple_of(base_e + k, 8)
                pltpu.sync_copy(src_hbm.at[pl.ds(e0, CHUNK)], sidx_v)
                pltpu.sync_copy(dst_hbm.at[pl.ds(e0, CHUNK)], didx_v)
                pltpu.async_copy(table_hbm.at[sidx_v], rows_v, sem).wait()
                pltpu.sync_copy(rows_v, accum_sh.at[didx_v], add=True)

        def writeout(o_hbm):
            @pl.when(s < NSUB - 1)
            def _():
                pltpu.sync_copy(accum_sh.at[pl.ds(r0, R_BIG)],
                                o_hbm.at[pl.ds(r0, R_BIG)])

            @pl.when(s == NSUB - 1)
            def _():
                pltpu.sync_copy(accum_sh.at[pl.ds(r0, R_LAST)],
                                o_hbm.at[pl.ds(r0, R_LAST)])

        def two_passes(ta_hbm, oa_hbm, tb_hbm, ob_hbm):
            zero_slice()
            plsc.subcore_barrier()
            run_edges(ta_hbm)
            plsc.subcore_barrier()
            writeout(oa_hbm)
            zero_slice()
            plsc.subcore_barrier()
            run_edges(tb_hbm)
            plsc.subcore_barrier()
            writeout(ob_hbm)

        @pl.when(c == 0)
        def _():
            two_passes(x0_hbm, o0_hbm, x1_hbm, o1_hbm)

        @pl.when(c == 1)
        def _():
            two_passes(x2_hbm, o2_hbm, x3_hbm, o3_hbm)

    return seg_kernel(*xq, src, dst, zrows)


def kernel(features, preference, mlp_W, mlp_b, conv1_W, lin1_W, lin1_b,
           g1_W, g1_b, conv2_W, lin2_W, lin2_b, g2_W, g2_b,
           id_embedding, edge_index):
    src = edge_index[0]
    dst = edge_index[1]
    zrows = jnp.zeros((R_BIG, Q), jnp.float32)

    mlp_b2 = mlp_b.reshape(1, D)
    lin1_b2 = lin1_b.reshape(1, D)
    g1_b2 = g1_b.reshape(1, D)
    lin2_b2 = lin2_b.reshape(1, D)
    g2_b2 = g2_b.reshape(1, D)

    temp = _mm_bias(features, mlp_W, mlp_b2)
    x0 = jnp.concatenate([preference, temp], axis=0)

    *xq1, xhat1 = _layer_pre(x0, conv1_W, lin1_W, lin1_b2, id_embedding)
    seg1 = _segsum(xq1, src, dst, zrows)
    *xq2, xhat2 = _layer_mid(seg1, xhat1, id_embedding,
                             g1_W, g1_b2, conv2_W, lin2_W, lin2_b2)
    seg2 = _segsum(xq2, src, dst, zrows)
    return _layer_post(seg2, xhat2, g2_W, g2_b2)


# speedup vs baseline: 5.8153x; 1.0695x over previous
import functools

import jax
import jax.numpy as jnp
from jax import lax
from jax.experimental import pallas as pl
from jax.experimental.pallas import tpu as pltpu
from jax.experimental.pallas import tpu_sc as plsc

N_USERS = 25000
N_ITEMS = 25000
N_NODES = N_USERS + N_ITEMS
N_EDGES = 800000
D_FEAT = 128
D = 64
Q = 16

NSUB = 16
E_PER_SUB = N_EDGES // NSUB
CHUNK = 400
NCH = E_PER_SUB // CHUNK
R_BIG = 3128
R_LAST = N_NODES - (NSUB - 1) * R_BIG

BR = 5000


def _lrelu(x):
    return jnp.where(x > 0, x, 0.01 * x)


def _quarter_specs():
    return [pl.BlockSpec((BR, Q), lambda i: (i, 0)) for _ in range(4)]


def _quarter_shapes():
    return [jax.ShapeDtypeStruct((N_NODES, Q), jnp.float32) for _ in range(4)]


def _k1_body(f_ref, w_ref, b_ref, o_ref):
    o_ref[...] = jnp.dot(f_ref[...], w_ref[...],
                         preferred_element_type=jnp.float32) + b_ref[...]


def _mm_bias(features, mlp_W, mlp_b):
    return pl.pallas_call(
        _k1_body,
        grid=(N_ITEMS // BR,),
        in_specs=[
            pl.BlockSpec((BR, D_FEAT), lambda i: (i, 0)),
            pl.BlockSpec((D_FEAT, D), lambda i: (0, 0)),
            pl.BlockSpec((1, D), lambda i: (0, 0)),
        ],
        out_specs=pl.BlockSpec((BR, D), lambda i: (i, 0)),
        out_shape=jax.ShapeDtypeStruct((N_ITEMS, D), jnp.float32),
    )(features, mlp_W, mlp_b)


def _k2_body(x_ref, id_ref, cw_ref, lw_ref, lb_ref,
             q0_ref, q1_ref, q2_ref, q3_ref, xh_ref):
    x = x_ref[...]
    n = jnp.sqrt(jnp.sum(x * x, axis=1, keepdims=True))
    xn = x / jnp.maximum(n, 1e-12)
    xw = jnp.dot(xn, cw_ref[...], preferred_element_type=jnp.float32)
    q0_ref[...] = xw[:, 0 * Q:1 * Q]
    q1_ref[...] = xw[:, 1 * Q:2 * Q]
    q2_ref[...] = xw[:, 2 * Q:3 * Q]
    q3_ref[...] = xw[:, 3 * Q:4 * Q]
    xh_ref[...] = _lrelu(jnp.dot(xn, lw_ref[...],
                                 preferred_element_type=jnp.float32)
                         + lb_ref[...]) + id_ref[...]


def _layer_pre(x0, conv1_W, lin1_W, lin1_b, id_embedding):
    return pl.pallas_call(
        _k2_body,
        grid=(N_NODES // BR,),
        in_specs=[
            pl.BlockSpec((BR, D), lambda i: (i, 0)),
            pl.BlockSpec((BR, D), lambda i: (i, 0)),
            pl.BlockSpec((D, D), lambda i: (0, 0)),
            pl.BlockSpec((D, D), lambda i: (0, 0)),
            pl.BlockSpec((1, D), lambda i: (0, 0)),
        ],
        out_specs=_quarter_specs() + [pl.BlockSpec((BR, D), lambda i: (i, 0))],
        out_shape=_quarter_shapes() + [
            jax.ShapeDtypeStruct((N_NODES, D), jnp.float32)],
    )(x0, id_embedding, conv1_W, lin1_W, lin1_b)


def _k3_body(s0_ref, s1_ref, s2_ref, s3_ref, xh1_ref, id_ref, gw_ref, gb_ref,
             cw_ref, lw_ref, lb_ref, q0_ref, q1_ref, q2_ref, q3_ref, xh2_ref):
    h = _lrelu(jnp.concatenate(
        [s0_ref[...], s1_ref[...], s2_ref[...], s3_ref[...]], axis=1))
    x1 = _lrelu(jnp.dot(h, gw_ref[...], preferred_element_type=jnp.float32)
                + gb_ref[...] + xh1_ref[...])
    xw = jnp.dot(x1, cw_ref[...], preferred_element_type=jnp.float32)
    q0_ref[...] = xw[:, 0 * Q:1 * Q]
    q1_ref[...] = xw[:, 1 * Q:2 * Q]
    q2_ref[...] = xw[:, 2 * Q:3 * Q]
    q3_ref[...] = xw[:, 3 * Q:4 * Q]
    xh2_ref[...] = _lrelu(jnp.dot(x1, lw_ref[...],
                                  preferred_element_type=jnp.float32)
                          + lb_ref[...]) + id_ref[...]


def _layer_mid(segs, xhat1, id_embedding, g1_W, g1_b, conv2_W,
               lin2_W, lin2_b):
    return pl.pallas_call(
        _k3_body,
        grid=(N_NODES // BR,),
        in_specs=_quarter_specs() + [
            pl.BlockSpec((BR, D), lambda i: (i, 0)),
            pl.BlockSpec((BR, D), lambda i: (i, 0)),
            pl.BlockSpec((D, D), lambda i: (0, 0)),
            pl.BlockSpec((1, D), lambda i: (0, 0)),
            pl.BlockSpec((D, D), lambda i: (0, 0)),
            pl.BlockSpec((D, D), lambda i: (0, 0)),
            pl.BlockSpec((1, D), lambda i: (0, 0)),
        ],
        out_specs=_quarter_specs() + [pl.BlockSpec((BR, D), lambda i: (i, 0))],
        out_shape=_quarter_shapes() + [
            jax.ShapeDtypeStruct((N_NODES, D), jnp.float32)],
    )(*segs, xhat1, id_embedding, g1_W, g1_b, conv2_W, lin2_W, lin2_b)


def _k4_body(s0_ref, s1_ref, s2_ref, s3_ref, xh2_ref, gw_ref, gb_ref, o_ref):
    h = _lrelu(jnp.concatenate(
        [s0_ref[...], s1_ref[...], s2_ref[...], s3_ref[...]], axis=1))
    o_ref[...] = _lrelu(jnp.dot(h, gw_ref[...],
                                preferred_element_type=jnp.float32)
                        + gb_ref[...] + xh2_ref[...])


def _layer_post(segs, xhat2, g2_W, g2_b):
    return pl.pallas_call(
        _k4_body,
        grid=(N_NODES // BR,),
        in_specs=_quarter_specs() + [
            pl.BlockSpec((BR, D), lambda i: (i, 0)),
            pl.BlockSpec((D, D), lambda i: (0, 0)),
            pl.BlockSpec((1, D), lambda i: (0, 0)),
        ],
        out_specs=pl.BlockSpec((BR, D), lambda i: (i, 0)),
        out_shape=jax.ShapeDtypeStruct((N_NODES, D), jnp.float32),
    )(*segs, xhat2, g2_W, g2_b)


def _segsum(xq, src, dst, zrows):
    mesh = plsc.VectorSubcoreMesh(core_axis_name="c", subcore_axis_name="s")

    @functools.partial(
        pl.kernel,
        out_type=tuple(jax.ShapeDtypeStruct((N_NODES, Q), jnp.float32)
                       for _ in range(4)),
        mesh=mesh,
        scratch_types=[
            pltpu.VMEM((CHUNK,), jnp.int32),
            pltpu.VMEM((CHUNK,), jnp.int32),
            pltpu.VMEM((CHUNK,), jnp.int32),
            pltpu.VMEM((CHUNK,), jnp.int32),
            pltpu.VMEM((CHUNK, Q), jnp.float32),
            pltpu.VMEM((CHUNK, Q), jnp.float32),
            pltpu.VMEM_SHARED((N_NODES, Q), jnp.float32),
            pltpu.SemaphoreType.DMA,
            pltpu.SemaphoreType.DMA,
        ],
        compiler_params=pltpu.CompilerParams(use_tc_tiling_on_sc=False),
    )
    def seg_kernel(x0_hbm, x1_hbm, x2_hbm, x3_hbm, src_hbm, dst_hbm, z_hbm,
                   o0_hbm, o1_hbm, o2_hbm, o3_hbm,
                   sidx0_v, sidx1_v, didx0_v, didx1_v, rows0_v, rows1_v,
                   accum_sh, gsem0, gsem1):
        c = lax.axis_index("c")
        s = lax.axis_index("s")
        r0 = pl.multiple_of(s * R_BIG, 8)
        sidx = (sidx0_v, sidx1_v)
        didx = (didx0_v, didx1_v)
        rows = (rows0_v, rows1_v)
        gsem = (gsem0, gsem1)

        def zero_slice():
            @pl.when(s < NSUB - 1)
            def _():
                pltpu.sync_copy(z_hbm, accum_sh.at[pl.ds(r0, R_BIG)])

            @pl.when(s == NSUB - 1)
            def _():
                pltpu.sync_copy(z_hbm.at[pl.ds(0, R_LAST)],
                                accum_sh.at[pl.ds(r0, R_LAST)])

        def run_edges(table_hbm):
            base_e = pl.multiple_of(s * E_PER_SUB, 8)

            def load_idx(k, b):
                e0 = pl.multiple_of(base_e + k * CHUNK, 8)
                pltpu.sync_copy(src_hbm.at[pl.ds(e0, CHUNK)], sidx[b])
                pltpu.sync_copy(dst_hbm.at[pl.ds(e0, CHUNK)], didx[b])

            def start_gather(b):
                pltpu.async_copy(table_hbm.at[sidx[b]], rows[b], gsem[b])

            def wait_gather(b):
                pltpu.make_async_copy(table_hbm.at[sidx[b]], rows[b],
                                      gsem[b]).wait()

            def scatter(b):
                pltpu.sync_copy(rows[b], accum_sh.at[didx[b]], add=True)

            load_idx(0, 0)
            start_gather(0)
            load_idx(1, 1)

            @pl.loop(0, NCH - 1, step=2)
            def _(j):
                for b in (0, 1):
                    k = j + b
                    wait_gather(b)
                    start_gather(1 - b)
                    scatter(b)

                    @pl.when(k + 2 < NCH)
                    def _():
                        load_idx(k + 2, b)

            wait_gather(0)
            scatter(0)

        def writeout(o_hbm):
            @pl.when(s < NSUB - 1)
            def _():
                pltpu.sync_copy(accum_sh.at[pl.ds(r0, R_BIG)],
                                o_hbm.at[pl.ds(r0, R_BIG)])

            @pl.when(s == NSUB - 1)
            def _():
                pltpu.sync_copy(accum_sh.at[pl.ds(r0, R_LAST)],
                                o_hbm.at[pl.ds(r0, R_LAST)])

        def two_passes(ta_hbm, oa_hbm, tb_hbm, ob_hbm):
            zero_slice()
            plsc.subcore_barrier()
            run_edges(ta_hbm)
            plsc.subcore_barrier()
            writeout(oa_hbm)
            zero_slice()
            plsc.subcore_barrier()
            run_edges(tb_hbm)
            plsc.subcore_barrier()
            writeout(ob_hbm)

        @pl.when(c == 0)
        def _():
            two_passes(x0_hbm, o0_hbm, x1_hbm, o1_hbm)

        @pl.when(c == 1)
        def _():
            two_passes(x2_hbm, o2_hbm, x3_hbm, o3_hbm)

    return seg_kernel(*xq, src, dst, zrows)


def kernel(features, preference, mlp_W, mlp_b, conv1_W, lin1_W, lin1_b,
           g1_W, g1_b, conv2_W, lin2_W, lin2_b, g2_W, g2_b,
           id_embedding, edge_index):
    src = edge_index[0]
    dst = edge_index[1]
    zrows = jnp.zeros((R_BIG, Q), jnp.float32)

    mlp_b2 = mlp_b.reshape(1, D)
    lin1_b2 = lin1_b.reshape(1, D)
    g1_b2 = g1_b.reshape(1, D)
    lin2_b2 = lin2_b.reshape(1, D)
    g2_b2 = g2_b.reshape(1, D)

    temp = _mm_bias(features, mlp_W, mlp_b2)
    x0 = jnp.concatenate([preference, temp], axis=0)

    *xq1, xhat1 = _layer_pre(x0, conv1_W, lin1_W, lin1_b2, id_embedding)
    seg1 = _segsum(xq1, src, dst, zrows)
    *xq2, xhat2 = _layer_mid(seg1, xhat1, id_embedding,
                             g1_W, g1_b2, conv2_W, lin2_W, lin2_b2)
    seg2 = _segsum(xq2, src, dst, zrows)
    return _layer_post(seg2, xhat2, g2_W, g2_b2)


# speedup vs baseline: 7.5985x; 1.3066x over previous
import functools

import jax
import jax.numpy as jnp
from jax import lax
from jax.experimental import pallas as pl
from jax.experimental.pallas import tpu as pltpu
from jax.experimental.pallas import tpu_sc as plsc

N_USERS = 25000
N_ITEMS = 25000
N_NODES = N_USERS + N_ITEMS
N_EDGES = 800000
D_FEAT = 128
D = 64
Q = 16
NPK = N_NODES // 8

NSUB = 16
E_PER_SUB = N_EDGES // NSUB
CHUNK = 400
NCH = E_PER_SUB // CHUNK
R_BIG = 3128
R_LAST = N_NODES - (NSUB - 1) * R_BIG

BR = 4096
PK = BR // 8


def _lrelu(x):
    return jnp.where(x > 0, x, 0.01 * x)


def _pack_quarters(xw, qrefs, scr_ref):
    scr_ref[...] = xw
    for s in range(8):
        row = scr_ref[pl.Slice(s, PK, 8), :]
        for qi, qref in enumerate(qrefs):
            qref[:, s * Q:(s + 1) * Q] = row[:, qi * Q:(qi + 1) * Q]


def _unpack_quarters(srefs, scr_ref):
    svs = [sref[...] for sref in srefs]
    for s in range(8):
        row = jnp.concatenate([sv[:, s * Q:(s + 1) * Q] for sv in svs],
                              axis=1)
        scr_ref[pl.Slice(s, PK, 8), :] = row
    return scr_ref[...]


def _grid(n):
    return (n + BR - 1) // BR


_PACK_SPEC = lambda: pl.BlockSpec((PK, 128), lambda i: (i, 0))


def _quarter_specs():
    return [_PACK_SPEC() for _ in range(4)]


def _quarter_shapes():
    return [jax.ShapeDtypeStruct((NPK, 128), jnp.float32) for _ in range(4)]


def _k1_body(f_ref, w_ref, b_ref, o_ref):
    o_ref[...] = jnp.dot(f_ref[...], w_ref[...],
                         preferred_element_type=jnp.float32) + b_ref[...]


def _mm_bias(features, mlp_W, mlp_b):
    return pl.pallas_call(
        _k1_body,
        grid=(_grid(N_ITEMS),),
        in_specs=[
            pl.BlockSpec((BR, D_FEAT), lambda i: (i, 0)),
            pl.BlockSpec((D_FEAT, D), lambda i: (0, 0)),
            pl.BlockSpec((1, D), lambda i: (0, 0)),
        ],
        out_specs=pl.BlockSpec((BR, D), lambda i: (i, 0)),
        out_shape=jax.ShapeDtypeStruct((N_ITEMS, D), jnp.float32),
    )(features, mlp_W, mlp_b)


def _k2_body(x_ref, id_ref, cw_ref, lw_ref, lb_ref,
             q0_ref, q1_ref, q2_ref, q3_ref, xh_ref, scr_ref):
    x = x_ref[...]
    n = jnp.sqrt(jnp.sum(x * x, axis=1, keepdims=True))
    xn = x / jnp.maximum(n, 1e-12)
    xw = jnp.dot(xn, cw_ref[...], preferred_element_type=jnp.float32)
    _pack_quarters(xw, (q0_ref, q1_ref, q2_ref, q3_ref), scr_ref)
    xh_ref[...] = _lrelu(jnp.dot(xn, lw_ref[...],
                                 preferred_element_type=jnp.float32)
                         + lb_ref[...]) + id_ref[...]


def _layer_pre(x0, conv1_W, lin1_W, lin1_b, id_embedding):
    return pl.pallas_call(
        _k2_body,
        grid=(_grid(N_NODES),),
        in_specs=[
            pl.BlockSpec((BR, D), lambda i: (i, 0)),
            pl.BlockSpec((BR, D), lambda i: (i, 0)),
            pl.BlockSpec((D, D), lambda i: (0, 0)),
            pl.BlockSpec((D, D), lambda i: (0, 0)),
            pl.BlockSpec((1, D), lambda i: (0, 0)),
        ],
        out_specs=_quarter_specs() + [pl.BlockSpec((BR, D), lambda i: (i, 0))],
        out_shape=_quarter_shapes() + [
            jax.ShapeDtypeStruct((N_NODES, D), jnp.float32)],
        scratch_shapes=[pltpu.VMEM((BR, D), jnp.float32)],
    )(x0, id_embedding, conv1_W, lin1_W, lin1_b)


def _k3_body(s0_ref, s1_ref, s2_ref, s3_ref, xh1_ref, id_ref, gw_ref, gb_ref,
             cw_ref, lw_ref, lb_ref, q0_ref, q1_ref, q2_ref, q3_ref, xh2_ref,
             scr_ref):
    h = _lrelu(_unpack_quarters((s0_ref, s1_ref, s2_ref, s3_ref), scr_ref))
    x1 = _lrelu(jnp.dot(h, gw_ref[...], preferred_element_type=jnp.float32)
                + gb_ref[...] + xh1_ref[...])
    xw = jnp.dot(x1, cw_ref[...], preferred_element_type=jnp.float32)
    _pack_quarters(xw, (q0_ref, q1_ref, q2_ref, q3_ref), scr_ref)
    xh2_ref[...] = _lrelu(jnp.dot(x1, lw_ref[...],
                                  preferred_element_type=jnp.float32)
                          + lb_ref[...]) + id_ref[...]


def _layer_mid(segs, xhat1, id_embedding, g1_W, g1_b, conv2_W,
               lin2_W, lin2_b):
    return pl.pallas_call(
        _k3_body,
        grid=(_grid(N_NODES),),
        in_specs=_quarter_specs() + [
            pl.BlockSpec((BR, D), lambda i: (i, 0)),
            pl.BlockSpec((BR, D), lambda i: (i, 0)),
            pl.BlockSpec((D, D), lambda i: (0, 0)),
            pl.BlockSpec((1, D), lambda i: (0, 0)),
            pl.BlockSpec((D, D), lambda i: (0, 0)),
            pl.BlockSpec((D, D), lambda i: (0, 0)),
            pl.BlockSpec((1, D), lambda i: (0, 0)),
        ],
        out_specs=_quarter_specs() + [pl.BlockSpec((BR, D), lambda i: (i, 0))],
        out_shape=_quarter_shapes() + [
            jax.ShapeDtypeStruct((N_NODES, D), jnp.float32)],
        scratch_shapes=[pltpu.VMEM((BR, D), jnp.float32)],
    )(*segs, xhat1, id_embedding, g1_W, g1_b, conv2_W, lin2_W, lin2_b)


def _k4_body(s0_ref, s1_ref, s2_ref, s3_ref, xh2_ref, gw_ref, gb_ref, o_ref,
             scr_ref):
    h = _lrelu(_unpack_quarters((s0_ref, s1_ref, s2_ref, s3_ref), scr_ref))
    o_ref[...] = _lrelu(jnp.dot(h, gw_ref[...],
                                preferred_element_type=jnp.float32)
                        + gb_ref[...] + xh2_ref[...])


def _layer_post(segs, xhat2, g2_W, g2_b):
    return pl.pallas_call(
        _k4_body,
        grid=(_grid(N_NODES),),
        in_specs=_quarter_specs() + [
            pl.BlockSpec((BR, D), lambda i: (i, 0)),
            pl.BlockSpec((D, D), lambda i: (0, 0)),
            pl.BlockSpec((1, D), lambda i: (0, 0)),
        ],
        out_specs=pl.BlockSpec((BR, D), lambda i: (i, 0)),
        out_shape=jax.ShapeDtypeStruct((N_NODES, D), jnp.float32),
        scratch_shapes=[pltpu.VMEM((BR, D), jnp.float32)],
    )(*segs, xhat2, g2_W, g2_b)


def _segsum(xq, src, dst, zrows):
    mesh = plsc.VectorSubcoreMesh(core_axis_name="c", subcore_axis_name="s")

    @functools.partial(
        pl.kernel,
        out_type=tuple(jax.ShapeDtypeStruct((N_NODES, Q), jnp.float32)
                       for _ in range(4)),
        mesh=mesh,
        scratch_types=[
            pltpu.VMEM((CHUNK,), jnp.int32),
            pltpu.VMEM((CHUNK,), jnp.int32),
            pltpu.VMEM((CHUNK,), jnp.int32),
            pltpu.VMEM((CHUNK,), jnp.int32),
            pltpu.VMEM((CHUNK, Q), jnp.float32),
            pltpu.VMEM((CHUNK, Q), jnp.float32),
            pltpu.VMEM_SHARED((N_NODES, Q), jnp.float32),
            pltpu.SemaphoreType.DMA,
            pltpu.SemaphoreType.DMA,
        ],
        compiler_params=pltpu.CompilerParams(use_tc_tiling_on_sc=False),
    )
    def seg_kernel(x0_hbm, x1_hbm, x2_hbm, x3_hbm, src_hbm, dst_hbm, z_hbm,
                   o0_hbm, o1_hbm, o2_hbm, o3_hbm,
                   sidx0_v, sidx1_v, didx0_v, didx1_v, rows0_v, rows1_v,
                   accum_sh, gsem0, gsem1):
        c = lax.axis_index("c")
        s = lax.axis_index("s")
        r0 = pl.multiple_of(s * R_BIG, 8)
        sidx = (sidx0_v, sidx1_v)
        didx = (didx0_v, didx1_v)
        rows = (rows0_v, rows1_v)
        gsem = (gsem0, gsem1)

        def zero_slice():
            @pl.when(s < NSUB - 1)
            def _():
                pltpu.sync_copy(z_hbm, accum_sh.at[pl.ds(r0, R_BIG)])

            @pl.when(s == NSUB - 1)
            def _():
                pltpu.sync_copy(z_hbm.at[pl.ds(0, R_LAST)],
                                accum_sh.at[pl.ds(r0, R_LAST)])

        def run_edges(table_hbm):
            base_e = pl.multiple_of(s * E_PER_SUB, 8)

            def load_idx(k, b):
                e0 = pl.multiple_of(base_e + k * CHUNK, 8)
                pltpu.sync_copy(src_hbm.at[pl.ds(e0, CHUNK)], sidx[b])
                pltpu.sync_copy(dst_hbm.at[pl.ds(e0, CHUNK)], didx[b])

            def start_gather(b):
                pltpu.async_copy(table_hbm.at[sidx[b]], rows[b], gsem[b])

            def wait_gather(b):
                pltpu.make_async_copy(table_hbm.at[sidx[b]], rows[b],
                                      gsem[b]).wait()

            def scatter(b):
                pltpu.sync_copy(rows[b], accum_sh.at[didx[b]], add=True)

            load_idx(0, 0)
            start_gather(0)
            load_idx(1, 1)

            @pl.loop(0, NCH - 1, step=2)
            def _(j):
                for b in (0, 1):
                    k = j + b
                    wait_gather(b)
                    start_gather(1 - b)
                    scatter(b)

                    @pl.when(k + 2 < NCH)
                    def _():
                        load_idx(k + 2, b)

            wait_gather(0)
            scatter(0)

        def writeout(o_hbm):
            @pl.when(s < NSUB - 1)
            def _():
                pltpu.sync_copy(accum_sh.at[pl.ds(r0, R_BIG)],
                                o_hbm.at[pl.ds(r0, R_BIG)])

            @pl.when(s == NSUB - 1)
            def _():
                pltpu.sync_copy(accum_sh.at[pl.ds(r0, R_LAST)],
                                o_hbm.at[pl.ds(r0, R_LAST)])

        def two_passes(ta_hbm, oa_hbm, tb_hbm, ob_hbm):
            zero_slice()
            plsc.subcore_barrier()
            run_edges(ta_hbm)
            plsc.subcore_barrier()
            writeout(oa_hbm)
            zero_slice()
            plsc.subcore_barrier()
            run_edges(tb_hbm)
            plsc.subcore_barrier()
            writeout(ob_hbm)

        @pl.when(c == 0)
        def _():
            two_passes(x0_hbm, o0_hbm, x1_hbm, o1_hbm)

        @pl.when(c == 1)
        def _():
            two_passes(x2_hbm, o2_hbm, x3_hbm, o3_hbm)

    return seg_kernel(*xq, src, dst, zrows)


def kernel(features, preference, mlp_W, mlp_b, conv1_W, lin1_W, lin1_b,
           g1_W, g1_b, conv2_W, lin2_W, lin2_b, g2_W, g2_b,
           id_embedding, edge_index):
    src = edge_index[0]
    dst = edge_index[1]
    zrows = jnp.zeros((R_BIG, Q), jnp.float32)

    mlp_b2 = mlp_b.reshape(1, D)
    lin1_b2 = lin1_b.reshape(1, D)
    g1_b2 = g1_b.reshape(1, D)
    lin2_b2 = lin2_b.reshape(1, D)
    g2_b2 = g2_b.reshape(1, D)

    def flat(q):
        return jnp.reshape(q, (N_NODES, Q))

    def packed(q):
        return jnp.reshape(q, (NPK, 128))

    temp = _mm_bias(features, mlp_W, mlp_b2)
    x0 = jnp.concatenate([preference, temp], axis=0)

    *xq1, xhat1 = _layer_pre(x0, conv1_W, lin1_W, lin1_b2, id_embedding)
    seg1 = _segsum([flat(q) for q in xq1], src, dst, zrows)
    *xq2, xhat2 = _layer_mid([packed(s) for s in seg1], xhat1, id_embedding,
                             g1_W, g1_b2, conv2_W, lin2_W, lin2_b2)
    seg2 = _segsum([flat(q) for q in xq2], src, dst, zrows)
    return _layer_post([packed(s) for s in seg2], xhat2, g2_W, g2_b2)


# speedup vs baseline: 8.0840x; 1.0639x over previous
import functools

import jax
import jax.numpy as jnp
from jax import lax
from jax.experimental import pallas as pl
from jax.experimental.pallas import tpu as pltpu
from jax.experimental.pallas import tpu_sc as plsc

N_USERS = 25000
N_ITEMS = 25000
N_NODES = N_USERS + N_ITEMS
N_EDGES = 800000
D_FEAT = 128
D = 64
Q = 16
NPK = N_NODES // 8

NSUB = 16
E_PER_SUB = N_EDGES // NSUB
CHUNK = 400
NCH = E_PER_SUB // CHUNK
R_BIG = 3128
R_LAST = N_NODES - (NSUB - 1) * R_BIG

BR = 4096
PK = BR // 8


def _lrelu(x):
    return jnp.where(x > 0, x, 0.01 * x)


def _pack_quarters(xw, qrefs, scr_ref):
    scr_ref[...] = xw
    for s in range(8):
        row = scr_ref[pl.Slice(s, PK, 8), :]
        for qi, qref in enumerate(qrefs):
            qref[:, s * Q:(s + 1) * Q] = row[:, qi * Q:(qi + 1) * Q]


def _unpack_quarters(srefs, scr_ref):
    svs = [sref[...] for sref in srefs]
    for s in range(8):
        row = jnp.concatenate([sv[:, s * Q:(s + 1) * Q] for sv in svs],
                              axis=1)
        scr_ref[pl.Slice(s, PK, 8), :] = row
    return scr_ref[...]


def _grid(n):
    return (n + BR - 1) // BR


_PACK_SPEC = lambda: pl.BlockSpec((PK, 128), lambda i: (i, 0))


def _quarter_specs():
    return [_PACK_SPEC() for _ in range(4)]


def _quarter_shapes():
    return [jax.ShapeDtypeStruct((NPK, 128), jnp.float32) for _ in range(4)]


def _k1_body(f_ref, w_ref, b_ref, o_ref):
    o_ref[...] = jnp.dot(f_ref[...], w_ref[...],
                         preferred_element_type=jnp.float32) + b_ref[...]


def _mm_bias(features, mlp_W, mlp_b):
    return pl.pallas_call(
        _k1_body,
        grid=(_grid(N_ITEMS),),
        in_specs=[
            pl.BlockSpec((BR, D_FEAT), lambda i: (i, 0)),
            pl.BlockSpec((D_FEAT, D), lambda i: (0, 0)),
            pl.BlockSpec((1, D), lambda i: (0, 0)),
        ],
        out_specs=pl.BlockSpec((BR, D), lambda i: (i, 0)),
        out_shape=jax.ShapeDtypeStruct((N_ITEMS, D), jnp.float32),
    )(features, mlp_W, mlp_b)


def _k2_body(x_ref, id_ref, cw_ref, lw_ref, lb_ref,
             q0_ref, q1_ref, q2_ref, q3_ref, xh_ref, scr_ref):
    x = x_ref[...]
    n = jnp.sqrt(jnp.sum(x * x, axis=1, keepdims=True))
    xn = x / jnp.maximum(n, 1e-12)
    xw = jnp.dot(xn, cw_ref[...], preferred_element_type=jnp.float32)
    _pack_quarters(xw, (q0_ref, q1_ref, q2_ref, q3_ref), scr_ref)
    xh_ref[...] = _lrelu(jnp.dot(xn, lw_ref[...],
                                 preferred_element_type=jnp.float32)
                         + lb_ref[...]) + id_ref[...]


def _layer_pre(x0, conv1_W, lin1_W, lin1_b, id_embedding):
    return pl.pallas_call(
        _k2_body,
        grid=(_grid(N_NODES),),
        in_specs=[
            pl.BlockSpec((BR, D), lambda i: (i, 0)),
            pl.BlockSpec((BR, D), lambda i: (i, 0)),
            pl.BlockSpec((D, D), lambda i: (0, 0)),
            pl.BlockSpec((D, D), lambda i: (0, 0)),
            pl.BlockSpec((1, D), lambda i: (0, 0)),
        ],
        out_specs=_quarter_specs() + [pl.BlockSpec((BR, D), lambda i: (i, 0))],
        out_shape=_quarter_shapes() + [
            jax.ShapeDtypeStruct((N_NODES, D), jnp.float32)],
        scratch_shapes=[pltpu.VMEM((BR, D), jnp.float32)],
    )(x0, id_embedding, conv1_W, lin1_W, lin1_b)


def _k3_body(s0_ref, s1_ref, s2_ref, s3_ref, xh1_ref, id_ref, gw_ref, gb_ref,
             cw_ref, lw_ref, lb_ref, q0_ref, q1_ref, q2_ref, q3_ref, xh2_ref,
             scr_ref):
    h = _lrelu(_unpack_quarters((s0_ref, s1_ref, s2_ref, s3_ref), scr_ref))
    x1 = _lrelu(jnp.dot(h, gw_ref[...], preferred_element_type=jnp.float32)
                + gb_ref[...] + xh1_ref[...])
    xw = jnp.dot(x1, cw_ref[...], preferred_element_type=jnp.float32)
    _pack_quarters(xw, (q0_ref, q1_ref, q2_ref, q3_ref), scr_ref)
    xh2_ref[...] = _lrelu(jnp.dot(x1, lw_ref[...],
                                  preferred_element_type=jnp.float32)
                          + lb_ref[...]) + id_ref[...]


def _layer_mid(segs, xhat1, id_embedding, g1_W, g1_b, conv2_W,
               lin2_W, lin2_b):
    return pl.pallas_call(
        _k3_body,
        grid=(_grid(N_NODES),),
        in_specs=_quarter_specs() + [
            pl.BlockSpec((BR, D), lambda i: (i, 0)),
            pl.BlockSpec((BR, D), lambda i: (i, 0)),
            pl.BlockSpec((D, D), lambda i: (0, 0)),
            pl.BlockSpec((1, D), lambda i: (0, 0)),
            pl.BlockSpec((D, D), lambda i: (0, 0)),
            pl.BlockSpec((D, D), lambda i: (0, 0)),
            pl.BlockSpec((1, D), lambda i: (0, 0)),
        ],
        out_specs=_quarter_specs() + [pl.BlockSpec((BR, D), lambda i: (i, 0))],
        out_shape=_quarter_shapes() + [
            jax.ShapeDtypeStruct((N_NODES, D), jnp.float32)],
        scratch_shapes=[pltpu.VMEM((BR, D), jnp.float32)],
    )(*segs, xhat1, id_embedding, g1_W, g1_b, conv2_W, lin2_W, lin2_b)


def _k4_body(s0_ref, s1_ref, s2_ref, s3_ref, xh2_ref, gw_ref, gb_ref, o_ref,
             scr_ref):
    h = _lrelu(_unpack_quarters((s0_ref, s1_ref, s2_ref, s3_ref), scr_ref))
    o_ref[...] = _lrelu(jnp.dot(h, gw_ref[...],
                                preferred_element_type=jnp.float32)
                        + gb_ref[...] + xh2_ref[...])


def _layer_post(segs, xhat2, g2_W, g2_b):
    return pl.pallas_call(
        _k4_body,
        grid=(_grid(N_NODES),),
        in_specs=_quarter_specs() + [
            pl.BlockSpec((BR, D), lambda i: (i, 0)),
            pl.BlockSpec((D, D), lambda i: (0, 0)),
            pl.BlockSpec((1, D), lambda i: (0, 0)),
        ],
        out_specs=pl.BlockSpec((BR, D), lambda i: (i, 0)),
        out_shape=jax.ShapeDtypeStruct((N_NODES, D), jnp.float32),
        scratch_shapes=[pltpu.VMEM((BR, D), jnp.float32)],
    )(*segs, xhat2, g2_W, g2_b)


def _segsum(xq, e3, zrows):
    mesh = plsc.VectorSubcoreMesh(core_axis_name="c", subcore_axis_name="s")

    @functools.partial(
        pl.kernel,
        out_type=tuple(jax.ShapeDtypeStruct((N_NODES, Q), jnp.float32)
                       for _ in range(4)),
        mesh=mesh,
        scratch_types=[
            pltpu.VMEM((2, CHUNK), jnp.int32),
            pltpu.VMEM((2, CHUNK), jnp.int32),
            pltpu.VMEM((2, CHUNK), jnp.int32),
            pltpu.VMEM((2, CHUNK), jnp.int32),
            pltpu.VMEM((CHUNK, Q), jnp.float32),
            pltpu.VMEM((CHUNK, Q), jnp.float32),
            pltpu.VMEM_SHARED((N_NODES, Q), jnp.float32),
            pltpu.SemaphoreType.DMA,
            pltpu.SemaphoreType.DMA,
            pltpu.SemaphoreType.DMA,
            pltpu.SemaphoreType.DMA,
            pltpu.SemaphoreType.DMA,
            pltpu.SemaphoreType.DMA,
            pltpu.SemaphoreType.DMA,
            pltpu.SemaphoreType.DMA,
        ],
        compiler_params=pltpu.CompilerParams(use_tc_tiling_on_sc=False),
    )
    def seg_kernel(x0_hbm, x1_hbm, x2_hbm, x3_hbm, e3_hbm, z_hbm,
                   o0_hbm, o1_hbm, o2_hbm, o3_hbm,
                   idx0_v, idx1_v, idx2_v, idx3_v, rows0_v, rows1_v,
                   accum_sh, gsem0, gsem1, isem0, isem1, isem2, isem3,
                   ssem0, ssem1):
        c = lax.axis_index("c")
        s = lax.axis_index("s")
        r0 = pl.multiple_of(s * R_BIG, 8)
        idx = (idx0_v, idx1_v, idx2_v, idx3_v)
        rows = (rows0_v, rows1_v)
        gsem = (gsem0, gsem1)
        isem = (isem0, isem1, isem2, isem3)
        ssem = (ssem0, ssem1)

        def zero_slice():
            @pl.when(s < NSUB - 1)
            def _():
                pltpu.sync_copy(z_hbm, accum_sh.at[pl.ds(r0, R_BIG)])

            @pl.when(s == NSUB - 1)
            def _():
                pltpu.sync_copy(z_hbm.at[pl.ds(0, R_LAST)],
                                accum_sh.at[pl.ds(r0, R_LAST)])

        def run_edges(table_hbm):
            base_c = s * NCH

            def start_idx(k, ib):
                pltpu.async_copy(e3_hbm.at[base_c + k], idx[ib], isem[ib])

            def wait_idx(ib):
                pltpu.make_async_copy(e3_hbm.at[base_c], idx[ib],
                                      isem[ib]).wait()

            def start_gather(rb, ib):
                pltpu.async_copy(table_hbm.at[idx[ib].at[0]], rows[rb],
                                 gsem[rb])

            def wait_gather(rb, ib):
                pltpu.make_async_copy(table_hbm.at[idx[ib].at[0]], rows[rb],
                                      gsem[rb]).wait()

            def start_scatter(rb, ib):
                pltpu.async_copy(rows[rb], accum_sh.at[idx[ib].at[1]],
                                 ssem[rb], add=True)

            def wait_scatter(rb):
                pltpu.make_async_copy(rows[rb], accum_sh.at[idx[0].at[1]],
                                      ssem[rb]).wait()

            start_idx(0, 0)
            start_idx(1, 1)
            wait_idx(0)
            start_gather(0, 0)

            @pl.loop(0, NCH - 1, step=4)
            def _(j):
                for b4 in range(4):
                    k = j + b4
                    rb = b4 % 2
                    ib = b4
                    ib1 = (b4 + 1) % 4
                    wait_gather(rb, ib)

                    @pl.when(k >= 1)
                    def _():
                        wait_scatter(1 - rb)
                    wait_idx(ib1)
                    start_gather(1 - rb, ib1)
                    start_scatter(rb, ib)

                    @pl.when(k + 2 < NCH)
                    def _():
                        start_idx(k + 2, (b4 + 2) % 4)

            wait_gather(0, 0)
            wait_scatter(1)
            start_scatter(0, 0)
            wait_scatter(0)

        def writeout(o_hbm):
            @pl.when(s < NSUB - 1)
            def _():
                pltpu.sync_copy(accum_sh.at[pl.ds(r0, R_BIG)],
                                o_hbm.at[pl.ds(r0, R_BIG)])

            @pl.when(s == NSUB - 1)
            def _():
                pltpu.sync_copy(accum_sh.at[pl.ds(r0, R_LAST)],
                                o_hbm.at[pl.ds(r0, R_LAST)])

        def two_passes(ta_hbm, oa_hbm, tb_hbm, ob_hbm):
            zero_slice()
            plsc.subcore_barrier()
            run_edges(ta_hbm)
            plsc.subcore_barrier()
            writeout(oa_hbm)
            zero_slice()
            plsc.subcore_barrier()
            run_edges(tb_hbm)
            plsc.subcore_barrier()
            writeout(ob_hbm)

        @pl.when(c == 0)
        def _():
            two_passes(x0_hbm, o0_hbm, x1_hbm, o1_hbm)

        @pl.when(c == 1)
        def _():
            two_passes(x2_hbm, o2_hbm, x3_hbm, o3_hbm)

    return seg_kernel(*xq, e3, zrows)


def kernel(features, preference, mlp_W, mlp_b, conv1_W, lin1_W, lin1_b,
           g1_W, g1_b, conv2_W, lin2_W, lin2_b, g2_W, g2_b,
           id_embedding, edge_index):
    e3 = jnp.stack([edge_index[0].reshape(NSUB * NCH, CHUNK),
                    edge_index[1].reshape(NSUB * NCH, CHUNK)], axis=1)
    zrows = jnp.zeros((R_BIG, Q), jnp.float32)

    mlp_b2 = mlp_b.reshape(1, D)
    lin1_b2 = lin1_b.reshape(1, D)
    g1_b2 = g1_b.reshape(1, D)
    lin2_b2 = lin2_b.reshape(1, D)
    g2_b2 = g2_b.reshape(1, D)

    def flat(q):
        return jnp.reshape(q, (N_NODES, Q))

    def packed(q):
        return jnp.reshape(q, (NPK, 128))

    temp = _mm_bias(features, mlp_W, mlp_b2)
    x0 = jnp.concatenate([preference, temp], axis=0)

    *xq1, xhat1 = _layer_pre(x0, conv1_W, lin1_W, lin1_b2, id_embedding)
    seg1 = _segsum([flat(q) for q in xq1], e3, zrows)
    *xq2, xhat2 = _layer_mid([packed(s) for s in seg1], xhat1, id_embedding,
                             g1_W, g1_b2, conv2_W, lin2_W, lin2_b2)
    seg2 = _segsum([flat(q) for q in xq2], e3, zrows)
    return _layer_post([packed(s) for s in seg2], xhat2, g2_W, g2_b2)


# speedup vs baseline: 8.6467x; 1.0696x over previous
import functools

import jax
import jax.numpy as jnp
from jax import lax
from jax.experimental import pallas as pl
from jax.experimental.pallas import tpu as pltpu
from jax.experimental.pallas import tpu_sc as plsc

N_USERS = 25000
N_ITEMS = 25000
N_NODES = N_USERS + N_ITEMS
N_EDGES = 800000
D_FEAT = 128
D = 64
Q = 16
NPK = N_NODES // 8

NSUB = 16
E_PER_SUB = N_EDGES // NSUB
CHUNK = 400
NCH = E_PER_SUB // CHUNK
R_BIG = 3128
R_LAST = N_NODES - (NSUB - 1) * R_BIG

BR = 4096
PK = BR // 8


def _lrelu(x):
    return jnp.where(x > 0, x, 0.01 * x)


def _pack_quarters(xw, qrefs, scr_ref):
    scr_ref[...] = xw
    for s in range(8):
        row = scr_ref[pl.Slice(s, PK, 8), :]
        for qi, qref in enumerate(qrefs):
            qref[:, s * Q:(s + 1) * Q] = row[:, qi * Q:(qi + 1) * Q]


def _unpack_quarters(srefs, scr_ref):
    svs = [sref[...] for sref in srefs]
    for s in range(8):
        row = jnp.concatenate([sv[:, s * Q:(s + 1) * Q] for sv in svs],
                              axis=1)
        scr_ref[pl.Slice(s, PK, 8), :] = row
    return scr_ref[...]


def _grid(n):
    return (n + BR - 1) // BR


_PACK_SPEC = lambda: pl.BlockSpec((PK, 128), lambda i: (i, 0))


def _quarter_specs():
    return [_PACK_SPEC() for _ in range(4)]


def _quarter_shapes():
    return [jax.ShapeDtypeStruct((NPK, 128), jnp.float32) for _ in range(4)]


BR1 = 1000
NB1 = N_USERS // BR1


def _k1_body(p_ref, f_ref, w_ref, b_ref, o_ref):
    i = pl.program_id(0)

    @pl.when(i < NB1)
    def _():
        o_ref[...] = p_ref[...]

    @pl.when(i >= NB1)
    def _():
        o_ref[...] = jnp.dot(f_ref[...], w_ref[...],
                             preferred_element_type=jnp.float32) + b_ref[...]


def _build_x0(preference, features, mlp_W, mlp_b):
    return pl.pallas_call(
        _k1_body,
        grid=(2 * NB1,),
        in_specs=[
            pl.BlockSpec((BR1, D), lambda i: (jnp.minimum(i, NB1 - 1), 0)),
            pl.BlockSpec((BR1, D_FEAT),
                         lambda i: (jnp.maximum(i - NB1, 0), 0)),
            pl.BlockSpec((D_FEAT, D), lambda i: (0, 0)),
            pl.BlockSpec((1, D), lambda i: (0, 0)),
        ],
        out_specs=pl.BlockSpec((BR1, D), lambda i: (i, 0)),
        out_shape=jax.ShapeDtypeStruct((N_NODES, D), jnp.float32),
    )(preference, features, mlp_W, mlp_b)


def _k2_body(x_ref, id_ref, cw_ref, lw_ref, lb_ref,
             q0_ref, q1_ref, q2_ref, q3_ref, xh_ref, scr_ref):
    x = x_ref[...]
    n = jnp.sqrt(jnp.sum(x * x, axis=1, keepdims=True))
    xn = x / jnp.maximum(n, 1e-12)
    xw = jnp.dot(xn, cw_ref[...], preferred_element_type=jnp.float32)
    _pack_quarters(xw, (q0_ref, q1_ref, q2_ref, q3_ref), scr_ref)
    xh_ref[...] = _lrelu(jnp.dot(xn, lw_ref[...],
                                 preferred_element_type=jnp.float32)
                         + lb_ref[...]) + id_ref[...]


def _layer_pre(x0, conv1_W, lin1_W, lin1_b, id_embedding):
    return pl.pallas_call(
        _k2_body,
        grid=(_grid(N_NODES),),
        in_specs=[
            pl.BlockSpec((BR, D), lambda i: (i, 0)),
            pl.BlockSpec((BR, D), lambda i: (i, 0)),
            pl.BlockSpec((D, D), lambda i: (0, 0)),
            pl.BlockSpec((D, D), lambda i: (0, 0)),
            pl.BlockSpec((1, D), lambda i: (0, 0)),
        ],
        out_specs=_quarter_specs() + [pl.BlockSpec((BR, D), lambda i: (i, 0))],
        out_shape=_quarter_shapes() + [
            jax.ShapeDtypeStruct((N_NODES, D), jnp.float32)],
        scratch_shapes=[pltpu.VMEM((BR, D), jnp.float32)],
    )(x0, id_embedding, conv1_W, lin1_W, lin1_b)


def _k3_body(s0_ref, s1_ref, s2_ref, s3_ref, xh1_ref, id_ref, gw_ref, gb_ref,
             cw_ref, lw_ref, lb_ref, q0_ref, q1_ref, q2_ref, q3_ref, xh2_ref,
             scr_ref):
    h = _lrelu(_unpack_quarters((s0_ref, s1_ref, s2_ref, s3_ref), scr_ref))
    x1 = _lrelu(jnp.dot(h, gw_ref[...], preferred_element_type=jnp.float32)
                + gb_ref[...] + xh1_ref[...])
    xw = jnp.dot(x1, cw_ref[...], preferred_element_type=jnp.float32)
    _pack_quarters(xw, (q0_ref, q1_ref, q2_ref, q3_ref), scr_ref)
    xh2_ref[...] = _lrelu(jnp.dot(x1, lw_ref[...],
                                  preferred_element_type=jnp.float32)
                          + lb_ref[...]) + id_ref[...]


def _layer_mid(segs, xhat1, id_embedding, g1_W, g1_b, conv2_W,
               lin2_W, lin2_b):
    return pl.pallas_call(
        _k3_body,
        grid=(_grid(N_NODES),),
        in_specs=_quarter_specs() + [
            pl.BlockSpec((BR, D), lambda i: (i, 0)),
            pl.BlockSpec((BR, D), lambda i: (i, 0)),
            pl.BlockSpec((D, D), lambda i: (0, 0)),
            pl.BlockSpec((1, D), lambda i: (0, 0)),
            pl.BlockSpec((D, D), lambda i: (0, 0)),
            pl.BlockSpec((D, D), lambda i: (0, 0)),
            pl.BlockSpec((1, D), lambda i: (0, 0)),
        ],
        out_specs=_quarter_specs() + [pl.BlockSpec((BR, D), lambda i: (i, 0))],
        out_shape=_quarter_shapes() + [
            jax.ShapeDtypeStruct((N_NODES, D), jnp.float32)],
        scratch_shapes=[pltpu.VMEM((BR, D), jnp.float32)],
    )(*segs, xhat1, id_embedding, g1_W, g1_b, conv2_W, lin2_W, lin2_b)


def _k4_body(s0_ref, s1_ref, s2_ref, s3_ref, xh2_ref, gw_ref, gb_ref, o_ref,
             scr_ref):
    h = _lrelu(_unpack_quarters((s0_ref, s1_ref, s2_ref, s3_ref), scr_ref))
    o_ref[...] = _lrelu(jnp.dot(h, gw_ref[...],
                                preferred_element_type=jnp.float32)
                        + gb_ref[...] + xh2_ref[...])


def _layer_post(segs, xhat2, g2_W, g2_b):
    return pl.pallas_call(
        _k4_body,
        grid=(_grid(N_NODES),),
        in_specs=_quarter_specs() + [
            pl.BlockSpec((BR, D), lambda i: (i, 0)),
            pl.BlockSpec((D, D), lambda i: (0, 0)),
            pl.BlockSpec((1, D), lambda i: (0, 0)),
        ],
        out_specs=pl.BlockSpec((BR, D), lambda i: (i, 0)),
        out_shape=jax.ShapeDtypeStruct((N_NODES, D), jnp.float32),
        scratch_shapes=[pltpu.VMEM((BR, D), jnp.float32)],
    )(*segs, xhat2, g2_W, g2_b)


def _segsum(xq, edge_index, zrows):
    mesh = plsc.VectorSubcoreMesh(core_axis_name="c", subcore_axis_name="s")

    @functools.partial(
        pl.kernel,
        out_type=tuple(jax.ShapeDtypeStruct((N_NODES, Q), jnp.float32)
                       for _ in range(4)),
        mesh=mesh,
        scratch_types=[
            pltpu.VMEM((CHUNK,), jnp.int32),
            pltpu.VMEM((CHUNK,), jnp.int32),
            pltpu.VMEM((CHUNK,), jnp.int32),
            pltpu.VMEM((CHUNK,), jnp.int32),
            pltpu.VMEM((CHUNK,), jnp.int32),
            pltpu.VMEM((CHUNK,), jnp.int32),
            pltpu.VMEM((CHUNK, Q), jnp.float32),
            pltpu.VMEM((CHUNK, Q), jnp.float32),
            pltpu.VMEM_SHARED((N_NODES, Q), jnp.float32),
            pltpu.SemaphoreType.DMA,
            pltpu.SemaphoreType.DMA,
            pltpu.SemaphoreType.DMA,
            pltpu.SemaphoreType.DMA,
            pltpu.SemaphoreType.DMA,
            pltpu.SemaphoreType.DMA,
            pltpu.SemaphoreType.DMA,
            pltpu.SemaphoreType.DMA,
            pltpu.SemaphoreType.DMA,
            pltpu.SemaphoreType.DMA,
        ],
        compiler_params=pltpu.CompilerParams(use_tc_tiling_on_sc=False),
    )
    def seg_kernel(x0_hbm, x1_hbm, x2_hbm, x3_hbm, e_hbm, z_hbm,
                   o0_hbm, o1_hbm, o2_hbm, o3_hbm,
                   sidx0_v, sidx1_v, didx0_v, didx1_v, didx2_v, didx3_v,
                   rows0_v, rows1_v, accum_sh,
                   gsem0, gsem1, ssem0, ssem1,
                   us0, us1, ud0, ud1, ud2, ud3):
        c = lax.axis_index("c")
        s = lax.axis_index("s")
        r0 = pl.multiple_of(s * R_BIG, 8)
        sidx = (sidx0_v, sidx1_v)
        didx = (didx0_v, didx1_v, didx2_v, didx3_v)
        rows = (rows0_v, rows1_v)
        gsem = (gsem0, gsem1)
        ssem = (ssem0, ssem1)
        isem_s = (us0, us1)
        isem_d = (ud0, ud1, ud2, ud3)

        def zero_slice():
            @pl.when(s < NSUB - 1)
            def _():
                pltpu.sync_copy(z_hbm, accum_sh.at[pl.ds(r0, R_BIG)])

            @pl.when(s == NSUB - 1)
            def _():
                pltpu.sync_copy(z_hbm.at[pl.ds(0, R_LAST)],
                                accum_sh.at[pl.ds(r0, R_LAST)])

        def run_edges(table_hbm):
            base_e = pl.multiple_of(s * E_PER_SUB, 8)

            def start_idx(k, sb, db):
                e0 = pl.multiple_of(base_e + k * CHUNK, 8)
                pltpu.async_copy(e_hbm.at[0, pl.ds(e0, CHUNK)], sidx[sb],
                                 isem_s[sb])
                pltpu.async_copy(e_hbm.at[1, pl.ds(e0, CHUNK)], didx[db],
                                 isem_d[db])

            def wait_idx(sb, db):
                pltpu.make_async_copy(e_hbm.at[0, pl.ds(base_e, CHUNK)],
                                      sidx[sb], isem_s[sb]).wait()
                pltpu.make_async_copy(e_hbm.at[1, pl.ds(base_e, CHUNK)],
                                      didx[db], isem_d[db]).wait()

            def start_gather(rb, sb):
                pltpu.async_copy(table_hbm.at[sidx[sb]], rows[rb], gsem[rb])

            def wait_gather(rb, sb):
                pltpu.make_async_copy(table_hbm.at[sidx[sb]], rows[rb],
                                      gsem[rb]).wait()

            def start_scatter(rb, db):
                pltpu.async_copy(rows[rb], accum_sh.at[didx[db]],
                                 ssem[rb], add=True)

            def wait_scatter(rb):
                pltpu.make_async_copy(rows[rb], accum_sh.at[didx[0]],
                                      ssem[rb]).wait()

            start_idx(0, 0, 0)
            start_idx(1, 1, 1)
            wait_idx(0, 0)
            start_gather(0, 0)

            @pl.loop(0, NCH - 1, step=4)
            def _(j):
                for b4 in range(4):
                    k = j + b4
                    rb = b4 % 2
                    db = b4
                    wait_gather(rb, rb)

                    @pl.when(k >= 1)
                    def _():
                        wait_scatter(1 - rb)
                    wait_idx(1 - rb, (b4 + 1) % 4)
                    start_gather(1 - rb, 1 - rb)
                    start_scatter(rb, db)

                    @pl.when(k + 2 < NCH)
                    def _():
                        start_idx(k + 2, rb, (b4 + 2) % 4)

            wait_gather(0, 0)
            wait_scatter(1)
            start_scatter(0, 0)
            wait_scatter(0)

        def writeout(o_hbm):
            @pl.when(s < NSUB - 1)
            def _():
                pltpu.sync_copy(accum_sh.at[pl.ds(r0, R_BIG)],
                                o_hbm.at[pl.ds(r0, R_BIG)])

            @pl.when(s == NSUB - 1)
            def _():
                pltpu.sync_copy(accum_sh.at[pl.ds(r0, R_LAST)],
                                o_hbm.at[pl.ds(r0, R_LAST)])

        def two_passes(ta_hbm, oa_hbm, tb_hbm, ob_hbm):
            zero_slice()
            plsc.subcore_barrier()
            run_edges(ta_hbm)
            plsc.subcore_barrier()
            writeout(oa_hbm)
            zero_slice()
            plsc.subcore_barrier()
            run_edges(tb_hbm)
            plsc.subcore_barrier()
            writeout(ob_hbm)

        @pl.when(c == 0)
        def _():
            two_passes(x0_hbm, o0_hbm, x1_hbm, o1_hbm)

        @pl.when(c == 1)
        def _():
            two_passes(x2_hbm, o2_hbm, x3_hbm, o3_hbm)

    return seg_kernel(*xq, edge_index, zrows)


def kernel(features, preference, mlp_W, mlp_b, conv1_W, lin1_W, lin1_b,
           g1_W, g1_b, conv2_W, lin2_W, lin2_b, g2_W, g2_b,
           id_embedding, edge_index):
    zrows = jnp.zeros((R_BIG, Q), jnp.float32)

    mlp_b2 = mlp_b.reshape(1, D)
    lin1_b2 = lin1_b.reshape(1, D)
    g1_b2 = g1_b.reshape(1, D)
    lin2_b2 = lin2_b.reshape(1, D)
    g2_b2 = g2_b.reshape(1, D)

    def flat(q):
        return jnp.reshape(q, (N_NODES, Q))

    def packed(q):
        return jnp.reshape(q, (NPK, 128))

    x0 = _build_x0(preference, features, mlp_W, mlp_b2)

    *xq1, xhat1 = _layer_pre(x0, conv1_W, lin1_W, lin1_b2, id_embedding)
    seg1 = _segsum([flat(q) for q in xq1], edge_index, zrows)
    *xq2, xhat2 = _layer_mid([packed(s) for s in seg1], xhat1, id_embedding,
                             g1_W, g1_b2, conv2_W, lin2_W, lin2_b2)
    seg2 = _segsum([flat(q) for q in xq2], edge_index, zrows)
    return _layer_post([packed(s) for s in seg2], xhat2, g2_W, g2_b2)


# speedup vs baseline: 9.7332x; 1.1257x over previous
import functools

import jax
import jax.numpy as jnp
from jax import lax
from jax.experimental import pallas as pl
from jax.experimental.pallas import tpu as pltpu
from jax.experimental.pallas import tpu_sc as plsc

N_USERS = 25000
N_ITEMS = 25000
N_NODES = N_USERS + N_ITEMS
N_EDGES = 800000
D_FEAT = 128
D = 64
Q = 16
NPK = N_NODES // 8

NSUB = 16
E_PER_SUB = N_EDGES // NSUB
CHUNK = 640
NCH = E_PER_SUB // CHUNK
TAIL = E_PER_SUB - NCH * CHUNK
MAIN = 4 * ((NCH - 1) // 4)
R_BIG = 3128
R_LAST = N_NODES - (NSUB - 1) * R_BIG

BR = 8192
PK = BR // 8


def _lrelu(x):
    return jnp.where(x > 0, x, 0.01 * x)


def _pack_quarters(xw, qrefs, scr_ref):
    scr_ref[...] = xw
    for s in range(8):
        row = scr_ref[pl.Slice(s, PK, 8), :]
        for qi, qref in enumerate(qrefs):
            qref[:, s * Q:(s + 1) * Q] = row[:, qi * Q:(qi + 1) * Q]


def _unpack_quarters(srefs, scr_ref):
    svs = [sref[...] for sref in srefs]
    for s in range(8):
        row = jnp.concatenate([sv[:, s * Q:(s + 1) * Q] for sv in svs],
                              axis=1)
        scr_ref[pl.Slice(s, PK, 8), :] = row
    return scr_ref[...]


def _grid(n):
    return (n + BR - 1) // BR


_PACK_SPEC = lambda: pl.BlockSpec((PK, 128), lambda i: (i, 0))


def _quarter_specs():
    return [_PACK_SPEC() for _ in range(4)]


def _quarter_shapes():
    return [jax.ShapeDtypeStruct((NPK, 128), jnp.float32) for _ in range(4)]


BR1 = 1000
NB1 = N_USERS // BR1


def _k1_body(p_ref, f_ref, w_ref, b_ref, o_ref):
    i = pl.program_id(0)

    @pl.when(i < NB1)
    def _():
        o_ref[...] = p_ref[...]

    @pl.when(i >= NB1)
    def _():
        o_ref[...] = jnp.dot(f_ref[...], w_ref[...],
                             preferred_element_type=jnp.float32) + b_ref[...]


def _build_x0(preference, features, mlp_W, mlp_b):
    return pl.pallas_call(
        _k1_body,
        grid=(2 * NB1,),
        in_specs=[
            pl.BlockSpec((BR1, D), lambda i: (jnp.minimum(i, NB1 - 1), 0)),
            pl.BlockSpec((BR1, D_FEAT),
                         lambda i: (jnp.maximum(i - NB1, 0), 0)),
            pl.BlockSpec((D_FEAT, D), lambda i: (0, 0)),
            pl.BlockSpec((1, D), lambda i: (0, 0)),
        ],
        out_specs=pl.BlockSpec((BR1, D), lambda i: (i, 0)),
        out_shape=jax.ShapeDtypeStruct((N_NODES, D), jnp.float32),
    )(preference, features, mlp_W, mlp_b)


def _k2_body(x_ref, id_ref, cw_ref, lw_ref, lb_ref,
             q0_ref, q1_ref, q2_ref, q3_ref, xh_ref, scr_ref):
    x = x_ref[...]
    n = jnp.sqrt(jnp.sum(x * x, axis=1, keepdims=True))
    xn = x / jnp.maximum(n, 1e-12)
    xw = jnp.dot(xn, cw_ref[...], preferred_element_type=jnp.float32)
    _pack_quarters(xw, (q0_ref, q1_ref, q2_ref, q3_ref), scr_ref)
    xh_ref[...] = _lrelu(jnp.dot(xn, lw_ref[...],
                                 preferred_element_type=jnp.float32)
                         + lb_ref[...]) + id_ref[...]


def _layer_pre(x0, conv1_W, lin1_W, lin1_b, id_embedding):
    return pl.pallas_call(
        _k2_body,
        grid=(_grid(N_NODES),),
        in_specs=[
            pl.BlockSpec((BR, D), lambda i: (i, 0)),
            pl.BlockSpec((BR, D), lambda i: (i, 0)),
            pl.BlockSpec((D, D), lambda i: (0, 0)),
            pl.BlockSpec((D, D), lambda i: (0, 0)),
            pl.BlockSpec((1, D), lambda i: (0, 0)),
        ],
        out_specs=_quarter_specs() + [pl.BlockSpec((BR, D), lambda i: (i, 0))],
        out_shape=_quarter_shapes() + [
            jax.ShapeDtypeStruct((N_NODES, D), jnp.float32)],
        scratch_shapes=[pltpu.VMEM((BR, D), jnp.float32)],
    )(x0, id_embedding, conv1_W, lin1_W, lin1_b)


def _k3_body(s0_ref, s1_ref, s2_ref, s3_ref, xh1_ref, id_ref, gw_ref, gb_ref,
             cw_ref, lw_ref, lb_ref, q0_ref, q1_ref, q2_ref, q3_ref, xh2_ref,
             scr_ref):
    h = _lrelu(_unpack_quarters((s0_ref, s1_ref, s2_ref, s3_ref), scr_ref))
    x1 = _lrelu(jnp.dot(h, gw_ref[...], preferred_element_type=jnp.float32)
                + gb_ref[...] + xh1_ref[...])
    xw = jnp.dot(x1, cw_ref[...], preferred_element_type=jnp.float32)
    _pack_quarters(xw, (q0_ref, q1_ref, q2_ref, q3_ref), scr_ref)
    xh2_ref[...] = _lrelu(jnp.dot(x1, lw_ref[...],
                                  preferred_element_type=jnp.float32)
                          + lb_ref[...]) + id_ref[...]


def _layer_mid(segs, xhat1, id_embedding, g1_W, g1_b, conv2_W,
               lin2_W, lin2_b):
    return pl.pallas_call(
        _k3_body,
        grid=(_grid(N_NODES),),
        in_specs=_quarter_specs() + [
            pl.BlockSpec((BR, D), lambda i: (i, 0)),
            pl.BlockSpec((BR, D), lambda i: (i, 0)),
            pl.BlockSpec((D, D), lambda i: (0, 0)),
            pl.BlockSpec((1, D), lambda i: (0, 0)),
            pl.BlockSpec((D, D), lambda i: (0, 0)),
            pl.BlockSpec((D, D), lambda i: (0, 0)),
            pl.BlockSpec((1, D), lambda i: (0, 0)),
        ],
        out_specs=_quarter_specs() + [pl.BlockSpec((BR, D), lambda i: (i, 0))],
        out_shape=_quarter_shapes() + [
            jax.ShapeDtypeStruct((N_NODES, D), jnp.float32)],
        scratch_shapes=[pltpu.VMEM((BR, D), jnp.float32)],
    )(*segs, xhat1, id_embedding, g1_W, g1_b, conv2_W, lin2_W, lin2_b)


def _k4_body(s0_ref, s1_ref, s2_ref, s3_ref, xh2_ref, gw_ref, gb_ref, o_ref,
             scr_ref):
    h = _lrelu(_unpack_quarters((s0_ref, s1_ref, s2_ref, s3_ref), scr_ref))
    o_ref[...] = _lrelu(jnp.dot(h, gw_ref[...],
                                preferred_element_type=jnp.float32)
                        + gb_ref[...] + xh2_ref[...])


def _layer_post(segs, xhat2, g2_W, g2_b):
    return pl.pallas_call(
        _k4_body,
        grid=(_grid(N_NODES),),
        in_specs=_quarter_specs() + [
            pl.BlockSpec((BR, D), lambda i: (i, 0)),
            pl.BlockSpec((D, D), lambda i: (0, 0)),
            pl.BlockSpec((1, D), lambda i: (0, 0)),
        ],
        out_specs=pl.BlockSpec((BR, D), lambda i: (i, 0)),
        out_shape=jax.ShapeDtypeStruct((N_NODES, D), jnp.float32),
        scratch_shapes=[pltpu.VMEM((BR, D), jnp.float32)],
    )(*segs, xhat2, g2_W, g2_b)


def _segsum(xq, edge_index, zrows):
    mesh = plsc.VectorSubcoreMesh(core_axis_name="c", subcore_axis_name="s")

    @functools.partial(
        pl.kernel,
        out_type=tuple(jax.ShapeDtypeStruct((N_NODES, Q), jnp.float32)
                       for _ in range(4)),
        mesh=mesh,
        scratch_types=[
            pltpu.VMEM((CHUNK,), jnp.int32),
            pltpu.VMEM((CHUNK,), jnp.int32),
            pltpu.VMEM((CHUNK,), jnp.int32),
            pltpu.VMEM((CHUNK,), jnp.int32),
            pltpu.VMEM((CHUNK,), jnp.int32),
            pltpu.VMEM((CHUNK,), jnp.int32),
            pltpu.VMEM((CHUNK, Q), jnp.float32),
            pltpu.VMEM((CHUNK, Q), jnp.float32),
            pltpu.VMEM((TAIL,), jnp.int32),
            pltpu.VMEM((TAIL,), jnp.int32),
            pltpu.VMEM((TAIL, Q), jnp.float32),
            pltpu.VMEM_SHARED((N_NODES, Q), jnp.float32),
            pltpu.SemaphoreType.DMA,
            pltpu.SemaphoreType.DMA,
            pltpu.SemaphoreType.DMA,
            pltpu.SemaphoreType.DMA,
            pltpu.SemaphoreType.DMA,
            pltpu.SemaphoreType.DMA,
            pltpu.SemaphoreType.DMA,
            pltpu.SemaphoreType.DMA,
            pltpu.SemaphoreType.DMA,
            pltpu.SemaphoreType.DMA,
        ],
        compiler_params=pltpu.CompilerParams(use_tc_tiling_on_sc=False),
    )
    def seg_kernel(x0_hbm, x1_hbm, x2_hbm, x3_hbm, e_hbm, z_hbm,
                   o0_hbm, o1_hbm, o2_hbm, o3_hbm,
                   sidx0_v, sidx1_v, didx0_v, didx1_v, didx2_v, didx3_v,
                   rows0_v, rows1_v, sidxt_v, didxt_v, rowst_v, accum_sh,
                   gsem0, gsem1, ssem0, ssem1,
                   us0, us1, ud0, ud1, ud2, ud3):
        c = lax.axis_index("c")
        s = lax.axis_index("s")
        r0 = pl.multiple_of(s * R_BIG, 8)
        sidx = (sidx0_v, sidx1_v)
        didx = (didx0_v, didx1_v, didx2_v, didx3_v)
        rows = (rows0_v, rows1_v)
        gsem = (gsem0, gsem1)
        ssem = (ssem0, ssem1)
        isem_s = (us0, us1)
        isem_d = (ud0, ud1, ud2, ud3)

        def zero_slice():
            @pl.when(s < NSUB - 1)
            def _():
                pltpu.sync_copy(z_hbm, accum_sh.at[pl.ds(r0, R_BIG)])

            @pl.when(s == NSUB - 1)
            def _():
                pltpu.sync_copy(z_hbm.at[pl.ds(0, R_LAST)],
                                accum_sh.at[pl.ds(r0, R_LAST)])

        def run_edges(table_hbm):
            base_e = pl.multiple_of(s * E_PER_SUB, 8)

            def start_idx(k, sb, db):
                e0 = pl.multiple_of(base_e + k * CHUNK, 8)
                pltpu.async_copy(e_hbm.at[0, pl.ds(e0, CHUNK)], sidx[sb],
                                 isem_s[sb])
                pltpu.async_copy(e_hbm.at[1, pl.ds(e0, CHUNK)], didx[db],
                                 isem_d[db])

            def wait_idx(sb, db):
                pltpu.make_async_copy(e_hbm.at[0, pl.ds(base_e, CHUNK)],
                                      sidx[sb], isem_s[sb]).wait()
                pltpu.make_async_copy(e_hbm.at[1, pl.ds(base_e, CHUNK)],
                                      didx[db], isem_d[db]).wait()

            def start_gather(rb, sb):
                pltpu.async_copy(table_hbm.at[sidx[sb]], rows[rb], gsem[rb])

            def wait_gather(rb, sb):
                pltpu.make_async_copy(table_hbm.at[sidx[sb]], rows[rb],
                                      gsem[rb]).wait()

            def start_scatter(rb, db):
                pltpu.async_copy(rows[rb], accum_sh.at[didx[db]],
                                 ssem[rb], add=True)

            def wait_scatter(rb):
                pltpu.make_async_copy(rows[rb], accum_sh.at[didx[0]],
                                      ssem[rb]).wait()

            start_idx(0, 0, 0)
            start_idx(1, 1, 1)
            wait_idx(0, 0)
            start_gather(0, 0)

            @pl.loop(0, MAIN, step=4)
            def _(j):
                for b4 in range(4):
                    k = j + b4
                    rb = b4 % 2
                    db = b4
                    wait_gather(rb, rb)

                    @pl.when(k >= 1)
                    def _():
                        wait_scatter(1 - rb)
                    wait_idx(1 - rb, (b4 + 1) % 4)
                    start_gather(1 - rb, 1 - rb)
                    start_scatter(rb, db)

                    @pl.when(k + 2 < NCH)
                    def _():
                        start_idx(k + 2, rb, (b4 + 2) % 4)

            for k in range(MAIN, NCH):
                rb, db = k % 2, k % 4
                wait_gather(rb, rb)
                wait_scatter(1 - rb)
                if k + 1 < NCH:
                    wait_idx(1 - rb, (k + 1) % 4)
                    start_gather(1 - rb, 1 - rb)
                start_scatter(rb, db)
                if k + 2 < NCH:
                    start_idx(k + 2, rb, (k + 2) % 4)
            wait_scatter((NCH - 1) % 2)

            et = pl.multiple_of(base_e + NCH * CHUNK, 8)
            pltpu.sync_copy(e_hbm.at[0, pl.ds(et, TAIL)], sidxt_v)
            pltpu.sync_copy(e_hbm.at[1, pl.ds(et, TAIL)], didxt_v)
            pltpu.async_copy(table_hbm.at[sidxt_v], rowst_v, gsem[0]).wait()
            pltpu.sync_copy(rowst_v, accum_sh.at[didxt_v], add=True)

        def writeout(o_hbm):
            @pl.when(s < NSUB - 1)
            def _():
                pltpu.sync_copy(accum_sh.at[pl.ds(r0, R_BIG)],
                                o_hbm.at[pl.ds(r0, R_BIG)])

            @pl.when(s == NSUB - 1)
            def _():
                pltpu.sync_copy(accum_sh.at[pl.ds(r0, R_LAST)],
                                o_hbm.at[pl.ds(r0, R_LAST)])

        def two_passes(ta_hbm, oa_hbm, tb_hbm, ob_hbm):
            zero_slice()
            plsc.subcore_barrier()
            run_edges(ta_hbm)
            plsc.subcore_barrier()
            writeout(oa_hbm)
            zero_slice()
            plsc.subcore_barrier()
            run_edges(tb_hbm)
            plsc.subcore_barrier()
            writeout(ob_hbm)

        @pl.when(c == 0)
        def _():
            two_passes(x0_hbm, o0_hbm, x1_hbm, o1_hbm)

        @pl.when(c == 1)
        def _():
            two_passes(x2_hbm, o2_hbm, x3_hbm, o3_hbm)

    return seg_kernel(*xq, edge_index, zrows)


def kernel(features, preference, mlp_W, mlp_b, conv1_W, lin1_W, lin1_b,
           g1_W, g1_b, conv2_W, lin2_W, lin2_b, g2_W, g2_b,
           id_embedding, edge_index):
    zrows = jnp.zeros((R_BIG, Q), jnp.float32)

    mlp_b2 = mlp_b.reshape(1, D)
    lin1_b2 = lin1_b.reshape(1, D)
    g1_b2 = g1_b.reshape(1, D)
    lin2_b2 = lin2_b.reshape(1, D)
    g2_b2 = g2_b.reshape(1, D)

    def flat(q):
        return jnp.reshape(q, (N_NODES, Q))

    def packed(q):
        return jnp.reshape(q, (NPK, 128))

    x0 = _build_x0(preference, features, mlp_W, mlp_b2)

    *xq1, xhat1 = _layer_pre(x0, conv1_W, lin1_W, lin1_b2, id_embedding)
    seg1 = _segsum([flat(q) for q in xq1], edge_index, zrows)
    *xq2, xhat2 = _layer_mid([packed(s) for s in seg1], xhat1, id_embedding,
                             g1_W, g1_b2, conv2_W, lin2_W, lin2_b2)
    seg2 = _segsum([flat(q) for q in xq2], edge_index, zrows)
    return _layer_post([packed(s) for s in seg2], xhat2, g2_W, g2_b2)


# speedup vs baseline: 10.1290x; 1.0407x over previous
import functools

import jax
import jax.numpy as jnp
from jax import lax
from jax.experimental import pallas as pl
from jax.experimental.pallas import tpu as pltpu
from jax.experimental.pallas import tpu_sc as plsc

N_USERS = 25000
N_ITEMS = 25000
N_NODES = N_USERS + N_ITEMS
N_EDGES = 800000
D_FEAT = 128
D = 64
Q = 16
NPK = N_NODES // 8

NSUB = 16
E_PER_SUB = N_EDGES // NSUB
CHUNK = 640
NCH = E_PER_SUB // CHUNK
TAIL = E_PER_SUB - NCH * CHUNK
MAIN = 4 * ((NCH - 1) // 4)
R_BIG = 3128
R_LAST = N_NODES - (NSUB - 1) * R_BIG

BR = 4096
PK = BR // 8


def _lrelu(x):
    return jnp.where(x > 0, x, 0.01 * x)


def _pack_quarters(xw, qrefs, scr_ref):
    scr_ref[...] = xw
    for s in range(8):
        row = scr_ref[pl.Slice(s, PK, 8), :]
        for qi, qref in enumerate(qrefs):
            qref[:, s * Q:(s + 1) * Q] = row[:, qi * Q:(qi + 1) * Q]


def _unpack_quarters(srefs, scr_ref):
    svs = [sref[...] for sref in srefs]
    for s in range(8):
        row = jnp.concatenate([sv[:, s * Q:(s + 1) * Q] for sv in svs],
                              axis=1)
        scr_ref[pl.Slice(s, PK, 8), :] = row
    return scr_ref[...]


def _grid(n):
    return (n + BR - 1) // BR


_PACK_SPEC = lambda: pl.BlockSpec((PK, 128), lambda i: (i, 0))


def _quarter_specs():
    return [_PACK_SPEC() for _ in range(4)]


def _quarter_shapes():
    return [jax.ShapeDtypeStruct((NPK, 128), jnp.float32) for _ in range(4)]


BR1 = 5000
NB1 = N_USERS // BR1


def _k1_body(p_ref, f_ref, w_ref, b_ref, o_ref):
    i = pl.program_id(0)

    @pl.when(i < NB1)
    def _():
        o_ref[...] = p_ref[...]

    @pl.when(i >= NB1)
    def _():
        o_ref[...] = jnp.dot(f_ref[...], w_ref[...],
                             preferred_element_type=jnp.float32) + b_ref[...]


def _build_x0(preference, features, mlp_W, mlp_b):
    return pl.pallas_call(
        _k1_body,
        grid=(2 * NB1,),
        in_specs=[
            pl.BlockSpec((BR1, D), lambda i: (jnp.minimum(i, NB1 - 1), 0)),
            pl.BlockSpec((BR1, D_FEAT),
                         lambda i: (jnp.maximum(i - NB1, 0), 0)),
            pl.BlockSpec((D_FEAT, D), lambda i: (0, 0)),
            pl.BlockSpec((1, D), lambda i: (0, 0)),
        ],
        out_specs=pl.BlockSpec((BR1, D), lambda i: (i, 0)),
        out_shape=jax.ShapeDtypeStruct((N_NODES, D), jnp.float32),
        compiler_params=pltpu.CompilerParams(
            dimension_semantics=("parallel",)),
    )(preference, features, mlp_W, mlp_b)


def _k2_body(x_ref, id_ref, cw_ref, lw_ref, lb_ref,
             q0_ref, q1_ref, q2_ref, q3_ref, xh_ref, scr_ref):
    x = x_ref[...]
    n = jnp.sqrt(jnp.sum(x * x, axis=1, keepdims=True))
    xn = x / jnp.maximum(n, 1e-12)
    xw = jnp.dot(xn, cw_ref[...], preferred_element_type=jnp.float32)
    _pack_quarters(xw, (q0_ref, q1_ref, q2_ref, q3_ref), scr_ref)
    xh_ref[...] = _lrelu(jnp.dot(xn, lw_ref[...],
                                 preferred_element_type=jnp.float32)
                         + lb_ref[...]) + id_ref[...]


def _layer_pre(x0, conv1_W, lin1_W, lin1_b, id_embedding):
    return pl.pallas_call(
        _k2_body,
        grid=(_grid(N_NODES),),
        in_specs=[
            pl.BlockSpec((BR, D), lambda i: (i, 0)),
            pl.BlockSpec((BR, D), lambda i: (i, 0)),
            pl.BlockSpec((D, D), lambda i: (0, 0)),
            pl.BlockSpec((D, D), lambda i: (0, 0)),
            pl.BlockSpec((1, D), lambda i: (0, 0)),
        ],
        out_specs=_quarter_specs() + [pl.BlockSpec((BR, D), lambda i: (i, 0))],
        out_shape=_quarter_shapes() + [
            jax.ShapeDtypeStruct((N_NODES, D), jnp.float32)],
        scratch_shapes=[pltpu.VMEM((BR, D), jnp.float32)],
        compiler_params=pltpu.CompilerParams(
            dimension_semantics=("parallel",)),
    )(x0, id_embedding, conv1_W, lin1_W, lin1_b)


def _k3_body(s0_ref, s1_ref, s2_ref, s3_ref, xh1_ref, id_ref, gw_ref, gb_ref,
             cw_ref, lw_ref, lb_ref, q0_ref, q1_ref, q2_ref, q3_ref, xh2_ref,
             scr_ref):
    h = _lrelu(_unpack_quarters((s0_ref, s1_ref, s2_ref, s3_ref), scr_ref))
    x1 = _lrelu(jnp.dot(h, gw_ref[...], preferred_element_type=jnp.float32)
                + gb_ref[...] + xh1_ref[...])
    xw = jnp.dot(x1, cw_ref[...], preferred_element_type=jnp.float32)
    _pack_quarters(xw, (q0_ref, q1_ref, q2_ref, q3_ref), scr_ref)
    xh2_ref[...] = _lrelu(jnp.dot(x1, lw_ref[...],
                                  preferred_element_type=jnp.float32)
                          + lb_ref[...]) + id_ref[...]


def _layer_mid(segs, xhat1, id_embedding, g1_W, g1_b, conv2_W,
               lin2_W, lin2_b):
    return pl.pallas_call(
        _k3_body,
        grid=(_grid(N_NODES),),
        in_specs=_quarter_specs() + [
            pl.BlockSpec((BR, D), lambda i: (i, 0)),
            pl.BlockSpec((BR, D), lambda i: (i, 0)),
            pl.BlockSpec((D, D), lambda i: (0, 0)),
            pl.BlockSpec((1, D), lambda i: (0, 0)),
            pl.BlockSpec((D, D), lambda i: (0, 0)),
            pl.BlockSpec((D, D), lambda i: (0, 0)),
            pl.BlockSpec((1, D), lambda i: (0, 0)),
        ],
        out_specs=_quarter_specs() + [pl.BlockSpec((BR, D), lambda i: (i, 0))],
        out_shape=_quarter_shapes() + [
            jax.ShapeDtypeStruct((N_NODES, D), jnp.float32)],
        scratch_shapes=[pltpu.VMEM((BR, D), jnp.float32)],
        compiler_params=pltpu.CompilerParams(
            dimension_semantics=("parallel",)),
    )(*segs, xhat1, id_embedding, g1_W, g1_b, conv2_W, lin2_W, lin2_b)


def _k4_body(s0_ref, s1_ref, s2_ref, s3_ref, xh2_ref, gw_ref, gb_ref, o_ref,
             scr_ref):
    h = _lrelu(_unpack_quarters((s0_ref, s1_ref, s2_ref, s3_ref), scr_ref))
    o_ref[...] = _lrelu(jnp.dot(h, gw_ref[...],
                                preferred_element_type=jnp.float32)
                        + gb_ref[...] + xh2_ref[...])


def _layer_post(segs, xhat2, g2_W, g2_b):
    return pl.pallas_call(
        _k4_body,
        grid=(_grid(N_NODES),),
        in_specs=_quarter_specs() + [
            pl.BlockSpec((BR, D), lambda i: (i, 0)),
            pl.BlockSpec((D, D), lambda i: (0, 0)),
            pl.BlockSpec((1, D), lambda i: (0, 0)),
        ],
        out_specs=pl.BlockSpec((BR, D), lambda i: (i, 0)),
        out_shape=jax.ShapeDtypeStruct((N_NODES, D), jnp.float32),
        scratch_shapes=[pltpu.VMEM((BR, D), jnp.float32)],
        compiler_params=pltpu.CompilerParams(
            dimension_semantics=("parallel",)),
    )(*segs, xhat2, g2_W, g2_b)


def _segsum(xq, edge_index, zrows):
    mesh = plsc.VectorSubcoreMesh(core_axis_name="c", subcore_axis_name="s")

    @functools.partial(
        pl.kernel,
        out_type=tuple(jax.ShapeDtypeStruct((N_NODES, Q), jnp.float32)
                       for _ in range(4)),
        mesh=mesh,
        scratch_types=[
            pltpu.VMEM((CHUNK,), jnp.int32),
            pltpu.VMEM((CHUNK,), jnp.int32),
            pltpu.VMEM((CHUNK,), jnp.int32),
            pltpu.VMEM((CHUNK,), jnp.int32),
            pltpu.VMEM((CHUNK,), jnp.int32),
            pltpu.VMEM((CHUNK,), jnp.int32),
            pltpu.VMEM((CHUNK, Q), jnp.float32),
            pltpu.VMEM((CHUNK, Q), jnp.float32),
            pltpu.VMEM((TAIL,), jnp.int32),
            pltpu.VMEM((TAIL,), jnp.int32),
            pltpu.VMEM((TAIL, Q), jnp.float32),
            pltpu.VMEM_SHARED((N_NODES, Q), jnp.float32),
            pltpu.SemaphoreType.DMA,
            pltpu.SemaphoreType.DMA,
            pltpu.SemaphoreType.DMA,
            pltpu.SemaphoreType.DMA,
            pltpu.SemaphoreType.DMA,
            pltpu.SemaphoreType.DMA,
            pltpu.SemaphoreType.DMA,
            pltpu.SemaphoreType.DMA,
            pltpu.SemaphoreType.DMA,
            pltpu.SemaphoreType.DMA,
        ],
        compiler_params=pltpu.CompilerParams(use_tc_tiling_on_sc=False),
    )
    def seg_kernel(x0_hbm, x1_hbm, x2_hbm, x3_hbm, e_hbm, z_hbm,
                   o0_hbm, o1_hbm, o2_hbm, o3_hbm,
                   sidx0_v, sidx1_v, didx0_v, didx1_v, didx2_v, didx3_v,
                   rows0_v, rows1_v, sidxt_v, didxt_v, rowst_v, accum_sh,
                   gsem0, gsem1, ssem0, ssem1,
                   us0, us1, ud0, ud1, ud2, ud3):
        c = lax.axis_index("c")
        s = lax.axis_index("s")
        r0 = pl.multiple_of(s * R_BIG, 8)
        sidx = (sidx0_v, sidx1_v)
        didx = (didx0_v, didx1_v, didx2_v, didx3_v)
        rows = (rows0_v, rows1_v)
        gsem = (gsem0, gsem1)
        ssem = (ssem0, ssem1)
        isem_s = (us0, us1)
        isem_d = (ud0, ud1, ud2, ud3)

        def zero_slice():
            @pl.when(s < NSUB - 1)
            def _():
                pltpu.sync_copy(z_hbm, accum_sh.at[pl.ds(r0, R_BIG)])

            @pl.when(s == NSUB - 1)
            def _():
                pltpu.sync_copy(z_hbm.at[pl.ds(0, R_LAST)],
                                accum_sh.at[pl.ds(r0, R_LAST)])

        def run_edges(table_hbm):
            base_e = pl.multiple_of(s * E_PER_SUB, 8)

            def start_idx(k, sb, db):
                e0 = pl.multiple_of(base_e + k * CHUNK, 8)
                pltpu.async_copy(e_hbm.at[0, pl.ds(e0, CHUNK)], sidx[sb],
                                 isem_s[sb])
                pltpu.async_copy(e_hbm.at[1, pl.ds(e0, CHUNK)], didx[db],
                                 isem_d[db])

            def wait_idx(sb, db):
                pltpu.make_async_copy(e_hbm.at[0, pl.ds(base_e, CHUNK)],
                                      sidx[sb], isem_s[sb]).wait()
                pltpu.make_async_copy(e_hbm.at[1, pl.ds(base_e, CHUNK)],
                                      didx[db], isem_d[db]).wait()

            def start_gather(rb, sb):
                pltpu.async_copy(table_hbm.at[sidx[sb]], rows[rb], gsem[rb])

            def wait_gather(rb, sb):
                pltpu.make_async_copy(table_hbm.at[sidx[sb]], rows[rb],
                                      gsem[rb]).wait()

            def start_scatter(rb, db):
                pltpu.async_copy(rows[rb], accum_sh.at[didx[db]],
                                 ssem[rb], add=True)

            def wait_scatter(rb):
                pltpu.make_async_copy(rows[rb], accum_sh.at[didx[0]],
                                      ssem[rb]).wait()

            start_idx(0, 0, 0)
            start_idx(1, 1, 1)
            wait_idx(0, 0)
            start_gather(0, 0)

            @pl.loop(0, MAIN, step=4)
            def _(j):
                for b4 in range(4):
                    k = j + b4
                    rb = b4 % 2
                    db = b4
                    wait_gather(rb, rb)

                    @pl.when(k >= 1)
                    def _():
                        wait_scatter(1 - rb)
                    wait_idx(1 - rb, (b4 + 1) % 4)
                    start_gather(1 - rb, 1 - rb)
                    start_scatter(rb, db)

                    @pl.when(k + 2 < NCH)
                    def _():
                        start_idx(k + 2, rb, (b4 + 2) % 4)

            for k in range(MAIN, NCH):
                rb, db = k % 2, k % 4
                wait_gather(rb, rb)
                wait_scatter(1 - rb)
                if k + 1 < NCH:
                    wait_idx(1 - rb, (k + 1) % 4)
                    start_gather(1 - rb, 1 - rb)
                start_scatter(rb, db)
                if k + 2 < NCH:
                    start_idx(k + 2, rb, (k + 2) % 4)
            wait_scatter((NCH - 1) % 2)

            et = pl.multiple_of(base_e + NCH * CHUNK, 8)
            pltpu.sync_copy(e_hbm.at[0, pl.ds(et, TAIL)], sidxt_v)
            pltpu.sync_copy(e_hbm.at[1, pl.ds(et, TAIL)], didxt_v)
            pltpu.async_copy(table_hbm.at[sidxt_v], rowst_v, gsem[0]).wait()
            pltpu.sync_copy(rowst_v, accum_sh.at[didxt_v], add=True)

        def writeout(o_hbm):
            @pl.when(s < NSUB - 1)
            def _():
                pltpu.sync_copy(accum_sh.at[pl.ds(r0, R_BIG)],
                                o_hbm.at[pl.ds(r0, R_BIG)])

            @pl.when(s == NSUB - 1)
            def _():
                pltpu.sync_copy(accum_sh.at[pl.ds(r0, R_LAST)],
                                o_hbm.at[pl.ds(r0, R_LAST)])

        def two_passes(ta_hbm, oa_hbm, tb_hbm, ob_hbm):
            zero_slice()
            plsc.subcore_barrier()
            run_edges(ta_hbm)
            plsc.subcore_barrier()
            writeout(oa_hbm)
            zero_slice()
            plsc.subcore_barrier()
            run_edges(tb_hbm)
            plsc.subcore_barrier()
            writeout(ob_hbm)

        @pl.when(c == 0)
        def _():
            two_passes(x0_hbm, o0_hbm, x1_hbm, o1_hbm)

        @pl.when(c == 1)
        def _():
            two_passes(x2_hbm, o2_hbm, x3_hbm, o3_hbm)

    return seg_kernel(*xq, edge_index, zrows)


def kernel(features, preference, mlp_W, mlp_b, conv1_W, lin1_W, lin1_b,
           g1_W, g1_b, conv2_W, lin2_W, lin2_b, g2_W, g2_b,
           id_embedding, edge_index):
    zrows = jnp.zeros((R_BIG, Q), jnp.float32)

    mlp_b2 = mlp_b.reshape(1, D)
    lin1_b2 = lin1_b.reshape(1, D)
    g1_b2 = g1_b.reshape(1, D)
    lin2_b2 = lin2_b.reshape(1, D)
    g2_b2 = g2_b.reshape(1, D)

    def flat(q):
        return jnp.reshape(q, (N_NODES, Q))

    def packed(q):
        return jnp.reshape(q, (NPK, 128))

    x0 = _build_x0(preference, features, mlp_W, mlp_b2)

    *xq1, xhat1 = _layer_pre(x0, conv1_W, lin1_W, lin1_b2, id_embedding)
    seg1 = _segsum([flat(q) for q in xq1], edge_index, zrows)
    *xq2, xhat2 = _layer_mid([packed(s) for s in seg1], xhat1, id_embedding,
                             g1_W, g1_b2, conv2_W, lin2_W, lin2_b2)
    seg2 = _segsum([flat(q) for q in xq2], edge_index, zrows)
    return _layer_post([packed(s) for s in seg2], xhat2, g2_W, g2_b2)
